# Initial kernel scaffold; baseline (speedup 1.0000x reference)
import jax
import jax.numpy as jnp
from jax.experimental import pallas as pl


def kernel(x_user, x_item, edge_r0, edge_r1, edge_r2, W_emb_user, b_emb_user, W_emb_item, b_emb_item, basis, coeff, h_bias):
    raise NotImplementedError("write your pallas kernel here")


# speedup vs baseline: 2.5056x; 2.5056x over previous
import functools

import jax
import jax.numpy as jnp
from jax import lax
from jax.experimental import pallas as pl
from jax.experimental.pallas import tpu as pltpu
from jax.experimental.pallas import tpu_sc as plsc

N = 5000
D = 128
E = 100000
NPAD = 5120
NC = 2
NS = 16
NW = NC * NS
EP = 102400
TILE_E = EP // NW
GROUPS = TILE_E // 128
STRIPE = 32
NBLK = NPAD // 128
DEG_PER_SUB = NPAD // NS


def _sc_segment_sums(tables, srcs, dsts, zrow, zdeg, ones3):
    mesh = plsc.VectorSubcoreMesh(core_axis_name="c", subcore_axis_name="s")

    def body(tab_hbm, srcs_hbm, dsts_hbm,
             zrow_hbm, zdeg_hbm, ones3_hbm,
             s_out, deg_out,
             s_sh, deg_sh, idx_s, idx_d, rows, ones_v, sem):
        cid = lax.axis_index("c")
        sid = lax.axis_index("s")
        wid = sid * NC + cid

        pltpu.sync_copy(zdeg_hbm,
                        deg_sh.at[pl.ds(sid * DEG_PER_SUB, DEG_PER_SUB)])

        def relation(r, carry):
            for j in range(3):
                blk = sid + NS * j

                @pl.when(blk < NBLK)
                def _(blk=blk):
                    pltpu.sync_copy(zrow_hbm, s_sh.at[pl.ds(blk * 128, 128)])
            plsc.subcore_barrier()

            pltpu.sync_copy(ones3_hbm.at[r], ones_v)
            row0 = (r * NW + wid) * STRIPE
            pltpu.sync_copy(srcs_hbm.at[pl.ds(row0, STRIPE)], idx_s)
            pltpu.sync_copy(dsts_hbm.at[pl.ds(row0, STRIPE)], idx_d)

            def step(g, c):
                pltpu.async_copy(tab_hbm.at[idx_s.at[g]], rows, sem).wait()
                pltpu.sync_copy(rows, s_sh.at[idx_d.at[g]], add=True)
                pltpu.sync_copy(ones_v, deg_sh.at[idx_d.at[g]], add=True)
                return c

            lax.fori_loop(0, GROUPS, step, 0)
            plsc.subcore_barrier()

            for j in range(3):
                blk = sid + NS * j

                @pl.when(blk < NBLK)
                def _(blk=blk):
                    pltpu.sync_copy(s_sh.at[pl.ds(blk * 128, 128)],
                                    s_out.at[cid, r].at[pl.ds(blk * 128, 128)])
            plsc.subcore_barrier()
            return carry

        lax.fori_loop(0, 3, relation, 0)

        doff = sid * DEG_PER_SUB
        pltpu.sync_copy(deg_sh.at[pl.ds(doff, DEG_PER_SUB)],
                        deg_out.at[cid].at[pl.ds(doff, DEG_PER_SUB)])

    kern = pl.kernel(
        body,
        out_type=(
            jax.ShapeDtypeStruct((NC, 3, NPAD, D), jnp.float32),
            jax.ShapeDtypeStruct((NC, NPAD, 16), jnp.float32),
        ),
        mesh=mesh,
        scratch_types=[
            pltpu.VMEM_SHARED((NPAD, D), jnp.float32),
            pltpu.VMEM_SHARED((NPAD, 16), jnp.float32),
            pltpu.VMEM((STRIPE, 128), jnp.int32),
            pltpu.VMEM((STRIPE, 128), jnp.int32),
            pltpu.VMEM((128, D), jnp.float32),
            pltpu.VMEM((128, 16), jnp.float32),
            pltpu.SemaphoreType.DMA,
        ],
    )
    return kern(tables, srcs, dsts, zrow, zdeg, ones3)


def _tc_body(s_ref, deg_ref, wu_ref, wi_ref, bu_ref, bi_ref, basis_ref,
             hb_ref, coeff_ref, outu_ref, outi_ref):
    f32 = jnp.float32
    hi = jax.lax.Precision.HIGHEST

    def mm(a, b):
        return jax.lax.dot(a, b, precision=hi, preferred_element_type=f32)

    s0 = s_ref[0, 0] + s_ref[1, 0]
    s1 = s_ref[0, 1] + s_ref[1, 1]
    s2 = s_ref[0, 2] + s_ref[1, 2]
    deg = deg_ref[0] + deg_ref[1]

    b0 = basis_ref[0]
    b1 = basis_ref[1]
    w0 = coeff_ref[0, 0] * b0 + coeff_ref[0, 1] * b1
    w1 = coeff_ref[1, 0] * b0 + coeff_ref[1, 1] * b1
    w2 = coeff_ref[2, 0] * b0 + coeff_ref[2, 1] * b1
    m0 = mm(wu_ref[...], w0)
    m1 = mm(wi_ref[...], w1)
    m2 = mm(wu_ref[...], w2)
    b0v = mm(bu_ref[...], w0)
    b1v = mm(bi_ref[...], w1)
    b2v = mm(bu_ref[...], w2)

    d0 = deg[:, 0:1]
    d1 = deg[:, 1:2]
    d2 = deg[:, 2:3]
    agg_i = (mm(s0, m0) + d0 * b0v) / jnp.maximum(d0, 1.0)
    agg_u = ((mm(s1, m1) + d1 * b1v) / jnp.maximum(d1, 1.0)
             + (mm(s2, m2) + d2 * b2v) / jnp.maximum(d2, 1.0))
    hb = hb_ref[...]
    outi_ref[...] = jnp.maximum(agg_i + hb, 0.0)
    outu_ref[...] = jnp.maximum(agg_u + hb, 0.0)


def _tc_combine(s_part, deg_part, wu, wi, bu, bi, basis, hb, coeff):
    blk = 640
    grid = NPAD // blk
    full = lambda *_: (0, 0)
    outu, outi = pl.pallas_call(
        _tc_body,
        grid=(grid,),
        in_specs=[
            pl.BlockSpec((NC, 3, blk, D), lambda i: (0, 0, i, 0)),
            pl.BlockSpec((NC, blk, 16), lambda i: (0, i, 0)),
            pl.BlockSpec((D, D), full),
            pl.BlockSpec((D, D), full),
            pl.BlockSpec((1, D), full),
            pl.BlockSpec((1, D), full),
            pl.BlockSpec((2, D, D), lambda i: (0, 0, 0)),
            pl.BlockSpec((1, D), full),
            pl.BlockSpec(memory_space=pltpu.SMEM),
        ],
        out_specs=[
            pl.BlockSpec((blk, D), lambda i: (i, 0)),
            pl.BlockSpec((blk, D), lambda i: (i, 0)),
        ],
        out_shape=[
            jax.ShapeDtypeStruct((NPAD, D), jnp.float32),
            jax.ShapeDtypeStruct((NPAD, D), jnp.float32),
        ],
    )(s_part, deg_part, wu, wi, bu, bi, basis, hb, coeff)
    return outu, outi


@jax.jit
def kernel(x_user, x_item, edge_r0, edge_r1, edge_r2,
           W_emb_user, b_emb_user, W_emb_item, b_emb_item,
           basis, coeff, h_bias):
    pad = EP - E
    pad_src = jnp.zeros((pad,), jnp.int32)

    def prep(edge, roff):
        src = jnp.concatenate([edge[0] + roff * N, pad_src])
        dst = jnp.concatenate([edge[1], jnp.full((pad,), N, jnp.int32)])
        return src, dst

    s0, d0 = prep(edge_r0, 0)
    s1, d1 = prep(edge_r1, 1)
    s2, d2 = prep(edge_r2, 2)

    def stripes(a0, a1, a2):
        a = jnp.stack([a0, a1, a2]).reshape(3, NW, GROUPS, 128)
        a = jnp.pad(a, ((0, 0), (0, 0), (0, STRIPE - GROUPS), (0, 0)))
        return a.reshape(3 * NW * STRIPE, 128)

    srcs = stripes(s0, s1, s2)
    dsts = stripes(d0, d1, d2)

    zrow = jnp.zeros((128, D), jnp.float32)
    zdeg = jnp.zeros((DEG_PER_SUB, 16), jnp.float32)
    ones3 = jnp.zeros((3, 128, 16), jnp.float32)
    for r in range(3):
        ones3 = ones3.at[r, :, r].set(1.0)

    tables = jnp.concatenate([x_user, x_item, x_user], axis=0)
    s_part, deg_part = _sc_segment_sums(
        tables, srcs, dsts, zrow, zdeg, ones3)

    outu, outi = _tc_combine(
        s_part, deg_part, W_emb_user, W_emb_item,
        b_emb_user.reshape(1, D), b_emb_item.reshape(1, D),
        basis, h_bias.reshape(1, D), coeff)

    return jnp.concatenate([outu[:N], outi[:N]], axis=0)


# speedup vs baseline: 2.7215x; 1.0861x over previous
import functools

import jax
import jax.numpy as jnp
from jax import lax
from jax.experimental import pallas as pl
from jax.experimental.pallas import tpu as pltpu
from jax.experimental.pallas import tpu_sc as plsc

N = 5000
D = 128
E = 100000
NPAD = 5120
NC = 2
NS = 16
NW = NC * NS
EP = 102400
TILE_E = EP // NW
GROUPS = TILE_E // 128
STRIPE = 32
NBLK = NPAD // 128
DEG_PER_SUB = NPAD // NS


def _sc_segment_sums(tables, srcs, dsts, zrow, zdeg, ones3):
    mesh = plsc.VectorSubcoreMesh(core_axis_name="c", subcore_axis_name="s")

    def body(tab_hbm, srcs_hbm, dsts_hbm,
             zrow_hbm, zdeg_hbm, ones3_hbm,
             s_out, deg_out,
             s_sh, deg_sh, idx_s, idx_d, rows0, rows1, ones_v, sem0, sem1):
        cid = lax.axis_index("c")
        sid = lax.axis_index("s")
        wid = sid * NC + cid

        pltpu.sync_copy(zdeg_hbm,
                        deg_sh.at[pl.ds(sid * DEG_PER_SUB, DEG_PER_SUB)])

        def relation(r, carry):
            for j in range(3):
                blk = sid + NS * j

                @pl.when(blk < NBLK)
                def _(blk=blk):
                    pltpu.sync_copy(zrow_hbm, s_sh.at[pl.ds(blk * 128, 128)])
            plsc.subcore_barrier()

            pltpu.sync_copy(ones3_hbm.at[r], ones_v)
            row0 = (r * NW + wid) * STRIPE
            pltpu.sync_copy(srcs_hbm.at[pl.ds(row0, STRIPE)], idx_s)
            pltpu.sync_copy(dsts_hbm.at[pl.ds(row0, STRIPE)], idx_d)

            pltpu.async_copy(tab_hbm.at[idx_s.at[0]], rows0, sem0)

            def step(i, c):
                g0 = 2 * i
                pltpu.make_async_copy(tab_hbm.at[idx_s.at[g0]], rows0,
                                      sem0).wait()
                pltpu.async_copy(tab_hbm.at[idx_s.at[g0 + 1]], rows1, sem1)
                pltpu.sync_copy(rows0, s_sh.at[idx_d.at[g0]], add=True)
                pltpu.sync_copy(ones_v, deg_sh.at[idx_d.at[g0]], add=True)
                pltpu.make_async_copy(tab_hbm.at[idx_s.at[g0 + 1]], rows1,
                                      sem1).wait()
                pltpu.async_copy(tab_hbm.at[idx_s.at[g0 + 2]], rows0, sem0)
                pltpu.sync_copy(rows1, s_sh.at[idx_d.at[g0 + 1]], add=True)
                pltpu.sync_copy(ones_v, deg_sh.at[idx_d.at[g0 + 1]], add=True)
                return c

            lax.fori_loop(0, (GROUPS - 1) // 2, step, 0)
            last = GROUPS - 1
            pltpu.make_async_copy(tab_hbm.at[idx_s.at[last]], rows0,
                                  sem0).wait()
            pltpu.sync_copy(rows0, s_sh.at[idx_d.at[last]], add=True)
            pltpu.sync_copy(ones_v, deg_sh.at[idx_d.at[last]], add=True)
            plsc.subcore_barrier()

            for j in range(3):
                blk = sid + NS * j

                @pl.when(blk < NBLK)
                def _(blk=blk):
                    pltpu.sync_copy(s_sh.at[pl.ds(blk * 128, 128)],
                                    s_out.at[cid, r].at[pl.ds(blk * 128, 128)])
            plsc.subcore_barrier()
            return carry

        lax.fori_loop(0, 3, relation, 0)

        doff = sid * DEG_PER_SUB
        pltpu.sync_copy(deg_sh.at[pl.ds(doff, DEG_PER_SUB)],
                        deg_out.at[cid].at[pl.ds(doff, DEG_PER_SUB)])

    kern = pl.kernel(
        body,
        out_type=(
            jax.ShapeDtypeStruct((NC, 3, NPAD, D), jnp.float32),
            jax.ShapeDtypeStruct((NC, NPAD, 16), jnp.float32),
        ),
        mesh=mesh,
        scratch_types=[
            pltpu.VMEM_SHARED((NPAD, D), jnp.float32),
            pltpu.VMEM_SHARED((NPAD, 16), jnp.float32),
            pltpu.VMEM((STRIPE, 128), jnp.int32),
            pltpu.VMEM((STRIPE, 128), jnp.int32),
            pltpu.VMEM((128, D), jnp.float32),
            pltpu.VMEM((128, D), jnp.float32),
            pltpu.VMEM((128, 16), jnp.float32),
            pltpu.SemaphoreType.DMA,
            pltpu.SemaphoreType.DMA,
        ],
    )
    return kern(tables, srcs, dsts, zrow, zdeg, ones3)


def _tc_body(s_ref, deg_ref, wu_ref, wi_ref, bu_ref, bi_ref, basis_ref,
             hb_ref, coeff_ref, outu_ref, outi_ref):
    f32 = jnp.float32
    hi = jax.lax.Precision.HIGHEST

    def mm(a, b):
        return jax.lax.dot(a, b, precision=hi, preferred_element_type=f32)

    s0 = s_ref[0, 0] + s_ref[1, 0]
    s1 = s_ref[0, 1] + s_ref[1, 1]
    s2 = s_ref[0, 2] + s_ref[1, 2]
    deg = deg_ref[0] + deg_ref[1]

    b0 = basis_ref[0]
    b1 = basis_ref[1]
    w0 = coeff_ref[0, 0] * b0 + coeff_ref[0, 1] * b1
    w1 = coeff_ref[1, 0] * b0 + coeff_ref[1, 1] * b1
    w2 = coeff_ref[2, 0] * b0 + coeff_ref[2, 1] * b1
    m0 = mm(wu_ref[...], w0)
    m1 = mm(wi_ref[...], w1)
    m2 = mm(wu_ref[...], w2)
    b0v = mm(bu_ref[...], w0)
    b1v = mm(bi_ref[...], w1)
    b2v = mm(bu_ref[...], w2)

    d0 = deg[:, 0:1]
    d1 = deg[:, 1:2]
    d2 = deg[:, 2:3]
    agg_i = (mm(s0, m0) + d0 * b0v) / jnp.maximum(d0, 1.0)
    agg_u = ((mm(s1, m1) + d1 * b1v) / jnp.maximum(d1, 1.0)
             + (mm(s2, m2) + d2 * b2v) / jnp.maximum(d2, 1.0))
    hb = hb_ref[...]
    outi_ref[...] = jnp.maximum(agg_i + hb, 0.0)
    outu_ref[...] = jnp.maximum(agg_u + hb, 0.0)


def _tc_combine(s_part, deg_part, wu, wi, bu, bi, basis, hb, coeff):
    blk = 640
    grid = NPAD // blk
    full = lambda *_: (0, 0)
    outu, outi = pl.pallas_call(
        _tc_body,
        grid=(grid,),
        in_specs=[
            pl.BlockSpec((NC, 3, blk, D), lambda i: (0, 0, i, 0)),
            pl.BlockSpec((NC, blk, 16), lambda i: (0, i, 0)),
            pl.BlockSpec((D, D), full),
            pl.BlockSpec((D, D), full),
            pl.BlockSpec((1, D), full),
            pl.BlockSpec((1, D), full),
            pl.BlockSpec((2, D, D), lambda i: (0, 0, 0)),
            pl.BlockSpec((1, D), full),
            pl.BlockSpec(memory_space=pltpu.SMEM),
        ],
        out_specs=[
            pl.BlockSpec((blk, D), lambda i: (i, 0)),
            pl.BlockSpec((blk, D), lambda i: (i, 0)),
        ],
        out_shape=[
            jax.ShapeDtypeStruct((NPAD, D), jnp.float32),
            jax.ShapeDtypeStruct((NPAD, D), jnp.float32),
        ],
    )(s_part, deg_part, wu, wi, bu, bi, basis, hb, coeff)
    return outu, outi


@jax.jit
def kernel(x_user, x_item, edge_r0, edge_r1, edge_r2,
           W_emb_user, b_emb_user, W_emb_item, b_emb_item,
           basis, coeff, h_bias):
    pad = EP - E
    pad_src = jnp.zeros((pad,), jnp.int32)

    def prep(edge, roff):
        src = jnp.concatenate([edge[0] + roff * N, pad_src])
        dst = jnp.concatenate([edge[1], jnp.full((pad,), N, jnp.int32)])
        return src, dst

    s0, d0 = prep(edge_r0, 0)
    s1, d1 = prep(edge_r1, 1)
    s2, d2 = prep(edge_r2, 2)

    def stripes(a0, a1, a2):
        a = jnp.stack([a0, a1, a2]).reshape(3, NW, GROUPS, 128)
        a = jnp.pad(a, ((0, 0), (0, 0), (0, STRIPE - GROUPS), (0, 0)))
        return a.reshape(3 * NW * STRIPE, 128)

    srcs = stripes(s0, s1, s2)
    dsts = stripes(d0, d1, d2)

    zrow = jnp.zeros((128, D), jnp.float32)
    zdeg = jnp.zeros((DEG_PER_SUB, 16), jnp.float32)
    ones3 = jnp.zeros((3, 128, 16), jnp.float32)
    for r in range(3):
        ones3 = ones3.at[r, :, r].set(1.0)

    tables = jnp.concatenate([x_user, x_item, x_user], axis=0)
    s_part, deg_part = _sc_segment_sums(
        tables, srcs, dsts, zrow, zdeg, ones3)

    outu, outi = _tc_combine(
        s_part, deg_part, W_emb_user, W_emb_item,
        b_emb_user.reshape(1, D), b_emb_item.reshape(1, D),
        basis, h_bias.reshape(1, D), coeff)

    return jnp.concatenate([outu[:N], outi[:N]], axis=0)


# speedup vs baseline: 2.7253x; 1.0014x over previous
import functools

import jax
import jax.numpy as jnp
from jax import lax
from jax.experimental import pallas as pl
from jax.experimental.pallas import tpu as pltpu
from jax.experimental.pallas import tpu_sc as plsc

N = 5000
D = 128
E = 100000
NPAD = 5120
NC = 2
NS = 16
NW = NC * NS
EP = 102400
TILE_E = EP // NW
GROUPS = TILE_E // 128
STRIPE = 32
NBLK = NPAD // 128
DEG_PER_SUB = NPAD // NS


def _sc_segment_sums(tables, srcs, dsts, zrow, zdeg, ones3):
    mesh = plsc.VectorSubcoreMesh(core_axis_name="c", subcore_axis_name="s")

    def body(tab_hbm, srcs_hbm, dsts_hbm,
             zrow_hbm, zdeg_hbm, ones3_hbm,
             s_out, deg_out,
             s_sh, deg_sh, idx_s, idx_d, rows0, rows1, ones_v,
             sem0, sem1, semd):
        cid = lax.axis_index("c")
        sid = lax.axis_index("s")
        wid = sid * NC + cid

        pltpu.sync_copy(zdeg_hbm,
                        deg_sh.at[pl.ds(sid * DEG_PER_SUB, DEG_PER_SUB)])

        def relation(r, carry):
            for j in range(3):
                blk = sid + NS * j

                @pl.when(blk < NBLK)
                def _(blk=blk):
                    pltpu.sync_copy(zrow_hbm, s_sh.at[pl.ds(blk * 128, 128)])
            plsc.subcore_barrier()

            pltpu.sync_copy(ones3_hbm.at[r], ones_v)
            row0 = (r * NW + wid) * STRIPE
            pltpu.sync_copy(srcs_hbm.at[pl.ds(row0, STRIPE)], idx_s)
            pltpu.sync_copy(dsts_hbm.at[pl.ds(row0, STRIPE)], idx_d)

            pltpu.async_copy(tab_hbm.at[idx_s.at[0]], rows0, sem0)

            def step(i, c):
                g0 = 2 * i
                pltpu.make_async_copy(tab_hbm.at[idx_s.at[g0]], rows0,
                                      sem0).wait()
                pltpu.async_copy(tab_hbm.at[idx_s.at[g0 + 1]], rows1, sem1)
                pltpu.sync_copy(rows0, s_sh.at[idx_d.at[g0]], add=True)
                pltpu.async_copy(ones_v, deg_sh.at[idx_d.at[g0]], semd,
                                 add=True)
                pltpu.make_async_copy(tab_hbm.at[idx_s.at[g0 + 1]], rows1,
                                      sem1).wait()
                pltpu.async_copy(tab_hbm.at[idx_s.at[g0 + 2]], rows0, sem0)
                pltpu.sync_copy(rows1, s_sh.at[idx_d.at[g0 + 1]], add=True)
                pltpu.async_copy(ones_v, deg_sh.at[idx_d.at[g0 + 1]], semd,
                                 add=True)
                return c

            lax.fori_loop(0, (GROUPS - 1) // 2, step, 0)
            last = GROUPS - 1
            pltpu.make_async_copy(tab_hbm.at[idx_s.at[last]], rows0,
                                  sem0).wait()
            pltpu.sync_copy(rows0, s_sh.at[idx_d.at[last]], add=True)
            pltpu.async_copy(ones_v, deg_sh.at[idx_d.at[last]], semd,
                             add=True)

            def drain(g, c):
                pltpu.make_async_copy(ones_v, deg_sh.at[idx_d.at[g]],
                                      semd).wait()
                return c

            lax.fori_loop(0, GROUPS, drain, 0)
            plsc.subcore_barrier()

            for j in range(3):
                blk = sid + NS * j

                @pl.when(blk < NBLK)
                def _(blk=blk):
                    pltpu.sync_copy(s_sh.at[pl.ds(blk * 128, 128)],
                                    s_out.at[cid, r].at[pl.ds(blk * 128, 128)])
            plsc.subcore_barrier()
            return carry

        lax.fori_loop(0, 3, relation, 0)

        doff = sid * DEG_PER_SUB
        pltpu.sync_copy(deg_sh.at[pl.ds(doff, DEG_PER_SUB)],
                        deg_out.at[cid].at[pl.ds(doff, DEG_PER_SUB)])

    kern = pl.kernel(
        body,
        out_type=(
            jax.ShapeDtypeStruct((NC, 3, NPAD, D), jnp.float32),
            jax.ShapeDtypeStruct((NC, NPAD, 16), jnp.float32),
        ),
        mesh=mesh,
        scratch_types=[
            pltpu.VMEM_SHARED((NPAD, D), jnp.float32),
            pltpu.VMEM_SHARED((NPAD, 16), jnp.float32),
            pltpu.VMEM((STRIPE, 128), jnp.int32),
            pltpu.VMEM((STRIPE, 128), jnp.int32),
            pltpu.VMEM((128, D), jnp.float32),
            pltpu.VMEM((128, D), jnp.float32),
            pltpu.VMEM((128, 16), jnp.float32),
            pltpu.SemaphoreType.DMA,
            pltpu.SemaphoreType.DMA,
            pltpu.SemaphoreType.DMA,
        ],
    )
    return kern(tables, srcs, dsts, zrow, zdeg, ones3)


def _tc_body(s_ref, deg_ref, wu_ref, wi_ref, bu_ref, bi_ref, basis_ref,
             hb_ref, coeff_ref, outu_ref, outi_ref):
    f32 = jnp.float32
    hi = jax.lax.Precision.HIGHEST

    def mm(a, b):
        return jax.lax.dot(a, b, precision=hi, preferred_element_type=f32)

    s0 = s_ref[0, 0] + s_ref[1, 0]
    s1 = s_ref[0, 1] + s_ref[1, 1]
    s2 = s_ref[0, 2] + s_ref[1, 2]
    deg = deg_ref[0] + deg_ref[1]

    b0 = basis_ref[0]
    b1 = basis_ref[1]
    w0 = coeff_ref[0, 0] * b0 + coeff_ref[0, 1] * b1
    w1 = coeff_ref[1, 0] * b0 + coeff_ref[1, 1] * b1
    w2 = coeff_ref[2, 0] * b0 + coeff_ref[2, 1] * b1
    m0 = mm(wu_ref[...], w0)
    m1 = mm(wi_ref[...], w1)
    m2 = mm(wu_ref[...], w2)
    b0v = mm(bu_ref[...], w0)
    b1v = mm(bi_ref[...], w1)
    b2v = mm(bu_ref[...], w2)

    d0 = deg[:, 0:1]
    d1 = deg[:, 1:2]
    d2 = deg[:, 2:3]
    agg_i = (mm(s0, m0) + d0 * b0v) / jnp.maximum(d0, 1.0)
    agg_u = ((mm(s1, m1) + d1 * b1v) / jnp.maximum(d1, 1.0)
             + (mm(s2, m2) + d2 * b2v) / jnp.maximum(d2, 1.0))
    hb = hb_ref[...]
    outi_ref[...] = jnp.maximum(agg_i + hb, 0.0)
    outu_ref[...] = jnp.maximum(agg_u + hb, 0.0)


def _tc_combine(s_part, deg_part, wu, wi, bu, bi, basis, hb, coeff):
    blk = 640
    grid = NPAD // blk
    full = lambda *_: (0, 0)
    outu, outi = pl.pallas_call(
        _tc_body,
        grid=(grid,),
        in_specs=[
            pl.BlockSpec((NC, 3, blk, D), lambda i: (0, 0, i, 0)),
            pl.BlockSpec((NC, blk, 16), lambda i: (0, i, 0)),
            pl.BlockSpec((D, D), full),
            pl.BlockSpec((D, D), full),
            pl.BlockSpec((1, D), full),
            pl.BlockSpec((1, D), full),
            pl.BlockSpec((2, D, D), lambda i: (0, 0, 0)),
            pl.BlockSpec((1, D), full),
            pl.BlockSpec(memory_space=pltpu.SMEM),
        ],
        out_specs=[
            pl.BlockSpec((blk, D), lambda i: (i, 0)),
            pl.BlockSpec((blk, D), lambda i: (i, 0)),
        ],
        out_shape=[
            jax.ShapeDtypeStruct((NPAD, D), jnp.float32),
            jax.ShapeDtypeStruct((NPAD, D), jnp.float32),
        ],
    )(s_part, deg_part, wu, wi, bu, bi, basis, hb, coeff)
    return outu, outi


@jax.jit
def kernel(x_user, x_item, edge_r0, edge_r1, edge_r2,
           W_emb_user, b_emb_user, W_emb_item, b_emb_item,
           basis, coeff, h_bias):
    pad = EP - E
    pad_src = jnp.zeros((pad,), jnp.int32)

    def prep(edge, roff):
        src = jnp.concatenate([edge[0] + roff * N, pad_src])
        dst = jnp.concatenate([edge[1], jnp.full((pad,), N, jnp.int32)])
        return src, dst

    s0, d0 = prep(edge_r0, 0)
    s1, d1 = prep(edge_r1, 1)
    s2, d2 = prep(edge_r2, 2)

    def stripes(a0, a1, a2):
        a = jnp.stack([a0, a1, a2]).reshape(3, NW, GROUPS, 128)
        a = jnp.pad(a, ((0, 0), (0, 0), (0, STRIPE - GROUPS), (0, 0)))
        return a.reshape(3 * NW * STRIPE, 128)

    srcs = stripes(s0, s1, s2)
    dsts = stripes(d0, d1, d2)

    zrow = jnp.zeros((128, D), jnp.float32)
    zdeg = jnp.zeros((DEG_PER_SUB, 16), jnp.float32)
    ones3 = jnp.zeros((3, 128, 16), jnp.float32)
    for r in range(3):
        ones3 = ones3.at[r, :, r].set(1.0)

    tables = jnp.concatenate([x_user, x_item, x_user], axis=0)
    s_part, deg_part = _sc_segment_sums(
        tables, srcs, dsts, zrow, zdeg, ones3)

    outu, outi = _tc_combine(
        s_part, deg_part, W_emb_user, W_emb_item,
        b_emb_user.reshape(1, D), b_emb_item.reshape(1, D),
        basis, h_bias.reshape(1, D), coeff)

    return jnp.concatenate([outu[:N], outi[:N]], axis=0)


# speedup vs baseline: 2.7254x; 1.0001x over previous
import functools

import jax
import jax.numpy as jnp
from jax import lax
from jax.experimental import pallas as pl
from jax.experimental.pallas import tpu as pltpu
from jax.experimental.pallas import tpu_sc as plsc

N = 5000
D = 128
E = 100000
NPAD = 5120
NC = 2
NS = 16
NW = NC * NS
EP = 102400
TILE_E = EP // NW
GROUPS = TILE_E // 128
STRIPE = 32
NBLK = NPAD // 128
DEG_PER_SUB = NPAD // NS


def _sc_segment_sums(tables, srcs, dsts, zrow, zdeg, ones3):
    mesh = plsc.VectorSubcoreMesh(core_axis_name="c", subcore_axis_name="s")

    def body(tab_hbm, srcs_hbm, dsts_hbm,
             zrow_hbm, zdeg_hbm, ones3_hbm,
             s_out, deg_out,
             s_sh, deg_sh, idx_s, idx_d, rows0, rows1, ones_v,
             sem0, sem1, semd):
        cid = lax.axis_index("c")
        sid = lax.axis_index("s")
        wid = sid * NC + cid

        pltpu.sync_copy(zdeg_hbm,
                        deg_sh.at[pl.ds(sid * DEG_PER_SUB, DEG_PER_SUB)])

        def relation(r, carry):
            for j in range(3):
                blk = sid + NS * j

                @pl.when(blk < NBLK)
                def _(blk=blk):
                    pltpu.sync_copy(zrow_hbm, s_sh.at[pl.ds(blk * 128, 128)])
            plsc.subcore_barrier()

            pltpu.sync_copy(ones3_hbm.at[r], ones_v)
            row0 = (r * NW + wid) * STRIPE
            pltpu.sync_copy(srcs_hbm.at[pl.ds(row0, STRIPE)], idx_s)
            pltpu.sync_copy(dsts_hbm.at[pl.ds(row0, STRIPE)], idx_d)

            pltpu.async_copy(tab_hbm.at[idx_s.at[0]], rows0, sem0)

            def step(i, c):
                g0 = 2 * i
                pltpu.make_async_copy(tab_hbm.at[idx_s.at[g0]], rows0,
                                      sem0).wait()
                pltpu.async_copy(tab_hbm.at[idx_s.at[g0 + 1]], rows1, sem1)
                pltpu.sync_copy(rows0, s_sh.at[idx_d.at[g0]], add=True)
                pltpu.async_copy(ones_v, deg_sh.at[idx_d.at[g0]], semd,
                                 add=True)
                pltpu.make_async_copy(tab_hbm.at[idx_s.at[g0 + 1]], rows1,
                                      sem1).wait()
                pltpu.async_copy(tab_hbm.at[idx_s.at[g0 + 2]], rows0, sem0)
                pltpu.sync_copy(rows1, s_sh.at[idx_d.at[g0 + 1]], add=True)
                pltpu.async_copy(ones_v, deg_sh.at[idx_d.at[g0 + 1]], semd,
                                 add=True)
                return c

            lax.fori_loop(0, (GROUPS - 1) // 2, step, 0)
            last = GROUPS - 1
            pltpu.make_async_copy(tab_hbm.at[idx_s.at[last]], rows0,
                                  sem0).wait()
            pltpu.sync_copy(rows0, s_sh.at[idx_d.at[last]], add=True)
            pltpu.async_copy(ones_v, deg_sh.at[idx_d.at[last]], semd,
                             add=True)

            def drain(g, c):
                pltpu.make_async_copy(ones_v, deg_sh.at[idx_d.at[g]],
                                      semd).wait()
                return c

            lax.fori_loop(0, GROUPS, drain, 0)
            plsc.subcore_barrier()

            for j in range(3):
                blk = sid + NS * j

                @pl.when(blk < NBLK)
                def _(blk=blk):
                    pltpu.sync_copy(s_sh.at[pl.ds(blk * 128, 128)],
                                    s_out.at[cid, r].at[pl.ds(blk * 128, 128)])
            plsc.subcore_barrier()
            return carry

        lax.fori_loop(0, 3, relation, 0)

        doff = sid * DEG_PER_SUB
        pltpu.sync_copy(deg_sh.at[pl.ds(doff, DEG_PER_SUB)],
                        deg_out.at[cid].at[pl.ds(doff, DEG_PER_SUB)])

    kern = pl.kernel(
        body,
        out_type=(
            jax.ShapeDtypeStruct((NC, 3, NPAD, D), jnp.float32),
            jax.ShapeDtypeStruct((NC, NPAD, 8), jnp.float32),
        ),
        mesh=mesh,
        scratch_types=[
            pltpu.VMEM_SHARED((NPAD, D), jnp.float32),
            pltpu.VMEM_SHARED((NPAD, 8), jnp.float32),
            pltpu.VMEM((STRIPE, 128), jnp.int32),
            pltpu.VMEM((STRIPE, 128), jnp.int32),
            pltpu.VMEM((128, D), jnp.float32),
            pltpu.VMEM((128, D), jnp.float32),
            pltpu.VMEM((128, 8), jnp.float32),
            pltpu.SemaphoreType.DMA,
            pltpu.SemaphoreType.DMA,
            pltpu.SemaphoreType.DMA,
        ],
    )
    return kern(tables, srcs, dsts, zrow, zdeg, ones3)


def _tc_body(s_ref, deg_ref, wu_ref, wi_ref, bu_ref, bi_ref, basis_ref,
             hb_ref, coeff_ref, outu_ref, outi_ref):
    f32 = jnp.float32
    hi = jax.lax.Precision.HIGHEST

    def mm(a, b):
        return jax.lax.dot(a, b, precision=hi, preferred_element_type=f32)

    s0 = s_ref[0, 0] + s_ref[1, 0]
    s1 = s_ref[0, 1] + s_ref[1, 1]
    s2 = s_ref[0, 2] + s_ref[1, 2]
    deg = deg_ref[0] + deg_ref[1]

    b0 = basis_ref[0]
    b1 = basis_ref[1]
    w0 = coeff_ref[0, 0] * b0 + coeff_ref[0, 1] * b1
    w1 = coeff_ref[1, 0] * b0 + coeff_ref[1, 1] * b1
    w2 = coeff_ref[2, 0] * b0 + coeff_ref[2, 1] * b1
    m0 = mm(wu_ref[...], w0)
    m1 = mm(wi_ref[...], w1)
    m2 = mm(wu_ref[...], w2)
    b0v = mm(bu_ref[...], w0)
    b1v = mm(bi_ref[...], w1)
    b2v = mm(bu_ref[...], w2)

    d0 = deg[:, 0:1]
    d1 = deg[:, 1:2]
    d2 = deg[:, 2:3]
    agg_i = (mm(s0, m0) + d0 * b0v) / jnp.maximum(d0, 1.0)
    agg_u = ((mm(s1, m1) + d1 * b1v) / jnp.maximum(d1, 1.0)
             + (mm(s2, m2) + d2 * b2v) / jnp.maximum(d2, 1.0))
    hb = hb_ref[...]
    outi_ref[...] = jnp.maximum(agg_i + hb, 0.0)
    outu_ref[...] = jnp.maximum(agg_u + hb, 0.0)


def _tc_combine(s_part, deg_part, wu, wi, bu, bi, basis, hb, coeff):
    blk = 640
    grid = NPAD // blk
    full = lambda *_: (0, 0)
    outu, outi = pl.pallas_call(
        _tc_body,
        grid=(grid,),
        in_specs=[
            pl.BlockSpec((NC, 3, blk, D), lambda i: (0, 0, i, 0)),
            pl.BlockSpec((NC, blk, 8), lambda i: (0, i, 0)),
            pl.BlockSpec((D, D), full),
            pl.BlockSpec((D, D), full),
            pl.BlockSpec((1, D), full),
            pl.BlockSpec((1, D), full),
            pl.BlockSpec((2, D, D), lambda i: (0, 0, 0)),
            pl.BlockSpec((1, D), full),
            pl.BlockSpec(memory_space=pltpu.SMEM),
        ],
        out_specs=[
            pl.BlockSpec((blk, D), lambda i: (i, 0)),
            pl.BlockSpec((blk, D), lambda i: (i, 0)),
        ],
        out_shape=[
            jax.ShapeDtypeStruct((NPAD, D), jnp.float32),
            jax.ShapeDtypeStruct((NPAD, D), jnp.float32),
        ],
    )(s_part, deg_part, wu, wi, bu, bi, basis, hb, coeff)
    return outu, outi


@jax.jit
def kernel(x_user, x_item, edge_r0, edge_r1, edge_r2,
           W_emb_user, b_emb_user, W_emb_item, b_emb_item,
           basis, coeff, h_bias):
    pad = EP - E
    pad_src = jnp.zeros((pad,), jnp.int32)

    def prep(edge, roff):
        src = jnp.concatenate([edge[0] + roff * N, pad_src])
        dst = jnp.concatenate([edge[1], jnp.full((pad,), N, jnp.int32)])
        return src, dst

    s0, d0 = prep(edge_r0, 0)
    s1, d1 = prep(edge_r1, 1)
    s2, d2 = prep(edge_r2, 2)

    def stripes(a0, a1, a2):
        a = jnp.stack([a0, a1, a2]).reshape(3, NW, GROUPS, 128)
        a = jnp.pad(a, ((0, 0), (0, 0), (0, STRIPE - GROUPS), (0, 0)))
        return a.reshape(3 * NW * STRIPE, 128)

    srcs = stripes(s0, s1, s2)
    dsts = stripes(d0, d1, d2)

    zrow = jnp.zeros((128, D), jnp.float32)
    zdeg = jnp.zeros((DEG_PER_SUB, 8), jnp.float32)
    ones3 = jnp.zeros((3, 128, 8), jnp.float32)
    for r in range(3):
        ones3 = ones3.at[r, :, r].set(1.0)

    tables = jnp.concatenate([x_user, x_item, x_user], axis=0)
    s_part, deg_part = _sc_segment_sums(
        tables, srcs, dsts, zrow, zdeg, ones3)

    outu, outi = _tc_combine(
        s_part, deg_part, W_emb_user, W_emb_item,
        b_emb_user.reshape(1, D), b_emb_item.reshape(1, D),
        basis, h_bias.reshape(1, D), coeff)

    return jnp.concatenate([outu[:N], outi[:N]], axis=0)
